# Initial kernel scaffold; baseline (speedup 1.0000x reference)
import functools

import jax
import jax.numpy as jnp
from jax.experimental import pallas as pl

N = 10150
E = 324800
D = 128
H = 4
C = 32
HOPS = 3
B = 50
S = 203


def _seg_softmax_unnorm(e, dst, n):
    m = jax.ops.segment_max(e, dst, num_segments=n)
    m = jnp.where(jnp.isfinite(m), m, 0.0)
    a = jnp.exp(e - m[dst])
    s = jax.ops.segment_sum(a, dst, num_segments=n)
    return a / (s[dst] + 1e-16)


def _gat(x, W, asrc, adst, b, src, dst):
    h = (x @ W).reshape(N, H, C)
    al_s = (h * asrc[None]).sum(-1)
    al_d = (h * adst[None]).sum(-1)
    loop = jnp.arange(N, dtype=src.dtype)
    s2 = jnp.concatenate([src, loop])
    d2 = jnp.concatenate([dst, loop])
    e = jax.nn.leaky_relu(al_s[s2] + al_d[d2], 0.2)
    a = _seg_softmax_unnorm(e, d2, N)
    out = jax.ops.segment_sum(h[s2] * a[:, :, None], d2, num_segments=N)
    return out.reshape(N, D) + b


def _trans(x, Wq, bq, Wk, bk, Wv, bv, Ws, bs, src, dst):
    q = (x @ Wq + bq).reshape(N, H, C)
    k = (x @ Wk + bk).reshape(N, H, C)
    v = (x @ Wv + bv).reshape(N, H, C)
    e = (q[dst] * k[src]).sum(-1) / jnp.sqrt(jnp.float32(C))
    a = _seg_softmax_unnorm(e, dst, N)
    out = jax.ops.segment_sum(v[src] * a[:, :, None], dst, num_segments=N).reshape(N, D)
    return out + (x @ Ws + bs)


def _epilogue_body(x_ref, wh_ref, se_w1_ref, se_b1_ref, se_w2_ref, se_b2_ref,
                   wf_ref, bf_ref, ln_g_ref, ln_b_ref, out_ref):
    wh = wh_ref[...]
    xr = x_ref[...]
    pool = jnp.mean(wh, axis=0, keepdims=True)
    t1 = jax.nn.gelu(pool @ se_w1_ref[...] + se_b1_ref[...], approximate=False)
    se = jax.nn.sigmoid(t1 @ se_w2_ref[...] + se_b2_ref[...])
    whr = wh * se
    f = (xr @ wf_ref[0] + whr @ wf_ref[1]) + bf_ref[...]
    mu = jnp.mean(f, axis=-1, keepdims=True)
    var = jnp.mean((f - mu) ** 2, axis=-1, keepdims=True)
    f = (f - mu) * jax.lax.rsqrt(var + 1e-5) * ln_g_ref[...] + ln_b_ref[...]
    out_ref[...] = jax.nn.gelu(f, approximate=False) + xr


def _epilogue(x, wh, se_w1, se_b1, se_w2, se_b2, Wf, bf, ln_g, ln_b):
    wf2 = Wf.reshape(2, D, D)
    return pl.pallas_call(
        _epilogue_body,
        grid=(B,),
        in_specs=[
            pl.BlockSpec((S, D), lambda i: (i, 0)),
            pl.BlockSpec((S, D), lambda i: (i, 0)),
            pl.BlockSpec((D, D // 8), lambda i: (0, 0)),
            pl.BlockSpec((1, D // 8), lambda i: (0, 0)),
            pl.BlockSpec((D // 8, D), lambda i: (0, 0)),
            pl.BlockSpec((1, D), lambda i: (0, 0)),
            pl.BlockSpec((2, D, D), lambda i: (0, 0, 0)),
            pl.BlockSpec((1, D), lambda i: (0, 0)),
            pl.BlockSpec((1, D), lambda i: (0, 0)),
            pl.BlockSpec((1, D), lambda i: (0, 0)),
        ],
        out_specs=pl.BlockSpec((S, D), lambda i: (i, 0)),
        out_shape=jax.ShapeDtypeStruct((N, D), jnp.float32),
    )(x, wh, se_w1, se_b1.reshape(1, -1), se_w2, se_b2.reshape(1, -1),
      wf2, bf.reshape(1, -1), ln_g.reshape(1, -1), ln_b.reshape(1, -1))


def kernel(x, edge_index, Wg, att_src, att_dst, bg, Wq, bq, Wk, bk, Wv, bv,
           Ws, bs, hop_w, se_w1, se_b1, se_w2, se_b2, Wf, bf, ln_g, ln_b):
    src = edge_index[0]
    dst = edge_index[1]
    hops = []
    for i in range(HOPS):
        inp_h = x if i == 0 else hops[-1]
        g = _gat(inp_h, Wg[i], att_src[i], att_dst[i], bg[i], src, dst)
        t = _trans(inp_h, Wq[i], bq[i], Wk[i], bk[i], Wv[i], bv[i],
                   Ws[i], bs[i], src, dst)
        hops.append((g + t) / 2.0)
    w = jax.nn.softmax(hop_w)
    wh = w[0] * hops[0] + w[1] * hops[1] + w[2] * hops[2]
    return _epilogue(x, wh, se_w1, se_b1, se_w2, se_b2, Wf, bf, ln_g, ln_b)


# speedup vs baseline: 1.0003x; 1.0003x over previous
import functools

import jax
import jax.numpy as jnp
from jax.experimental import pallas as pl

N = 10150
E = 324800
D = 128
H = 4
C = 32
HOPS = 3
B = 50
S = 203


def _seg_softmax_unnorm(e, dst, n):
    m = jax.ops.segment_max(e, dst, num_segments=n)
    m = jnp.where(jnp.isfinite(m), m, 0.0)
    a = jnp.exp(e - m[dst])
    s = jax.ops.segment_sum(a, dst, num_segments=n)
    return a / (s[dst] + 1e-16)


def _gat(x, W, asrc, adst, b, src, dst):
    h = (x @ W).reshape(N, H, C)
    al_s = (h * asrc[None]).sum(-1)
    al_d = (h * adst[None]).sum(-1)
    loop = jnp.arange(N, dtype=src.dtype)
    s2 = jnp.concatenate([src, loop])
    d2 = jnp.concatenate([dst, loop])
    e = jax.nn.leaky_relu(al_s[s2] + al_d[d2], 0.2)
    a = _seg_softmax_unnorm(e, d2, N)
    out = jax.ops.segment_sum(h[s2] * a[:, :, None], d2, num_segments=N)
    return out.reshape(N, D) + b


def _trans(x, Wq, bq, Wk, bk, Wv, bv, Ws, bs, src, dst):
    q = (x @ Wq + bq).reshape(N, H, C)
    k = (x @ Wk + bk).reshape(N, H, C)
    v = (x @ Wv + bv).reshape(N, H, C)
    e = (q[dst] * k[src]).sum(-1) / jnp.sqrt(jnp.float32(C))
    a = _seg_softmax_unnorm(e, dst, N)
    out = jax.ops.segment_sum(v[src] * a[:, :, None], dst, num_segments=N).reshape(N, D)
    return out + (x @ Ws + bs)


def _gelu_exact(x):
    return 0.5 * x * (1.0 + jax.lax.erf(x * 0.7071067811865476))


def _epilogue_body(x_ref, wh_ref, se_w1_ref, se_b1_ref, se_w2_ref, se_b2_ref,
                   wf_ref, bf_ref, ln_g_ref, ln_b_ref, out_ref):
    wh = wh_ref[0]
    xr = x_ref[0]
    pool = jnp.mean(wh, axis=0, keepdims=True)
    t1 = _gelu_exact(pool @ se_w1_ref[...] + se_b1_ref[...])
    se = jax.nn.sigmoid(t1 @ se_w2_ref[...] + se_b2_ref[...])
    whr = wh * se
    f = (xr @ wf_ref[0] + whr @ wf_ref[1]) + bf_ref[...]
    mu = jnp.mean(f, axis=-1, keepdims=True)
    var = jnp.mean((f - mu) ** 2, axis=-1, keepdims=True)
    f = (f - mu) * jax.lax.rsqrt(var + 1e-5) * ln_g_ref[...] + ln_b_ref[...]
    out_ref[0] = _gelu_exact(f) + xr


def _epilogue(x, wh, se_w1, se_b1, se_w2, se_b2, Wf, bf, ln_g, ln_b):
    wf2 = Wf.reshape(2, D, D)
    out = pl.pallas_call(
        _epilogue_body,
        grid=(B,),
        in_specs=[
            pl.BlockSpec((1, S, D), lambda i: (i, 0, 0)),
            pl.BlockSpec((1, S, D), lambda i: (i, 0, 0)),
            pl.BlockSpec((D, D // 8), lambda i: (0, 0)),
            pl.BlockSpec((1, D // 8), lambda i: (0, 0)),
            pl.BlockSpec((D // 8, D), lambda i: (0, 0)),
            pl.BlockSpec((1, D), lambda i: (0, 0)),
            pl.BlockSpec((2, D, D), lambda i: (0, 0, 0)),
            pl.BlockSpec((1, D), lambda i: (0, 0)),
            pl.BlockSpec((1, D), lambda i: (0, 0)),
            pl.BlockSpec((1, D), lambda i: (0, 0)),
        ],
        out_specs=pl.BlockSpec((1, S, D), lambda i: (i, 0, 0)),
        out_shape=jax.ShapeDtypeStruct((B, S, D), jnp.float32),
    )(x.reshape(B, S, D), wh.reshape(B, S, D), se_w1, se_b1.reshape(1, -1),
      se_w2, se_b2.reshape(1, -1), wf2, bf.reshape(1, -1),
      ln_g.reshape(1, -1), ln_b.reshape(1, -1))
    return out.reshape(N, D)


def kernel(x, edge_index, Wg, att_src, att_dst, bg, Wq, bq, Wk, bk, Wv, bv,
           Ws, bs, hop_w, se_w1, se_b1, se_w2, se_b2, Wf, bf, ln_g, ln_b):
    src = edge_index[0]
    dst = edge_index[1]
    hops = []
    for i in range(HOPS):
        inp_h = x if i == 0 else hops[-1]
        g = _gat(inp_h, Wg[i], att_src[i], att_dst[i], bg[i], src, dst)
        t = _trans(inp_h, Wq[i], bq[i], Wk[i], bk[i], Wv[i], bv[i],
                   Ws[i], bs[i], src, dst)
        hops.append((g + t) / 2.0)
    w = jax.nn.softmax(hop_w)
    wh = w[0] * hops[0] + w[1] * hops[1] + w[2] * hops[2]
    return _epilogue(x, wh, se_w1, se_b1, se_w2, se_b2, Wf, bf, ln_g, ln_b)


# speedup vs baseline: 12.4005x; 12.3972x over previous
import functools

import jax
import jax.numpy as jnp
from jax import lax
from jax.experimental import pallas as pl
from jax.experimental.pallas import tpu as pltpu
from jax.experimental.pallas import tpu_sc as plsc

N = 10150
E = 324800
D = 128
H = 4
C = 32
HOPS = 3
B = 50
S = 203

NP = 10240
EP = 327680
EPC = EP // 2
EPT = EP // 32
CHUNK = 128
NCHUNK = EPT // CHUNK
RPT = NP // 16
SROWS = NP * H // 128


def _gelu_exact(x):
    return 0.5 * x * (1.0 + jax.lax.erf(x * 0.7071067811865476))


def _hop_pre_body(x_ref, w_ref, b_ref, asrc_ref, adst_ref, smask_ref,
                  h_ref, ald_ref, ckv_ref, dq_ref, skip_ref):
    xb = x_ref[...]
    t = jnp.dot(xb, w_ref[...], preferred_element_type=jnp.float32) + b_ref[...]
    h = t[:, 0:128]
    q = t[:, 128:256]
    k = t[:, 256:384]
    v = t[:, 384:512]
    sk = t[:, 512:640]
    als = jnp.dot(h * asrc_ref[...], smask_ref[...],
                  preferred_element_type=jnp.float32)
    ald = jnp.dot(h * adst_ref[...], smask_ref[...],
                  preferred_element_type=jnp.float32)
    z112 = jnp.zeros((128, 112), jnp.float32)
    h_ref[...] = jnp.concatenate([h, als, z112], axis=1)
    ald_ref[...] = jnp.concatenate([ald, z112], axis=1)
    ckv_ref[...] = jnp.concatenate([k, v], axis=1)
    dq_ref[...] = q
    skip_ref[...] = sk


def _hop_pre(xp, wcat, bcat, asrc_flat, adst_flat, smask):
    grid = (NP // 128,)
    return pl.pallas_call(
        _hop_pre_body,
        grid=grid,
        in_specs=[
            pl.BlockSpec((128, D), lambda i: (i, 0)),
            pl.BlockSpec((D, 5 * D), lambda i: (0, 0)),
            pl.BlockSpec((1, 5 * D), lambda i: (0, 0)),
            pl.BlockSpec((1, D), lambda i: (0, 0)),
            pl.BlockSpec((1, D), lambda i: (0, 0)),
            pl.BlockSpec((D, 16), lambda i: (0, 0)),
        ],
        out_specs=[
            pl.BlockSpec((128, 2 * D), lambda i: (i, 0)),
            pl.BlockSpec((128, D), lambda i: (i, 0)),
            pl.BlockSpec((128, 2 * D), lambda i: (i, 0)),
            pl.BlockSpec((128, D), lambda i: (i, 0)),
            pl.BlockSpec((128, D), lambda i: (i, 0)),
        ],
        out_shape=[
            jax.ShapeDtypeStruct((NP, 2 * D), jnp.float32),
            jax.ShapeDtypeStruct((NP, D), jnp.float32),
            jax.ShapeDtypeStruct((NP, 2 * D), jnp.float32),
            jax.ShapeDtypeStruct((NP, D), jnp.float32),
            jax.ShapeDtypeStruct((NP, D), jnp.float32),
        ],
    )(xp, wcat, bcat, asrc_flat, adst_flat, smask)


_MESH = plsc.VectorSubcoreMesh(core_axis_name="c", subcore_axis_name="s")
_SC_PARAMS = pltpu.CompilerParams(needs_layout_passes=False)

GCH = 64
TCH = 64
DCH = 128
EPTD = EP // 16


def _gat_edge_body(sidx_hbm, didx_hbm, h_hbm, ald_hbm, bounds_hbm,
                   zeros_v_hbm,
                   out_v_hbm, p_hbm,
                   sidx_v, didx_v, hrow_v, adrow_v, orow_v, prow_v,
                   bounds_v, acc_sh, sem1, sem2):
    cid = lax.axis_index("c")
    sid = lax.axis_index("s")
    pltpu.sync_copy(zeros_v_hbm, acc_sh.at[pl.ds(sid * RPT, RPT)])
    pltpu.sync_copy(bounds_hbm, bounds_v)
    plsc.subcore_barrier()

    def chunk(g, carry):
        base = cid * EPC + sid * EPT + g * GCH
        pltpu.sync_copy(sidx_hbm.at[pl.ds(base, GCH)], sidx_v)
        pltpu.sync_copy(didx_hbm.at[pl.ds(base, GCH)], didx_v)
        cp1 = pltpu.async_copy(h_hbm.at[sidx_v], hrow_v, sem1)
        cp2 = pltpu.async_copy(ald_hbm.at[didx_v], adrow_v, sem2)
        cp1.wait()
        cp2.wait()
        for sub in range(GCH // 16):
            b16 = lax.iota(jnp.int32, 16) + sub * 16
            z16i = jnp.zeros((16,), jnp.int32)
            for hh in range(H):
                colh = jnp.full((16,), hh, jnp.int32)
                a_s = plsc.load_gather(hrow_v, [b16, colh + 128])
                a_d = plsc.load_gather(adrow_v, [b16, colh])
                zz = a_s + a_d
                e = jnp.where(zz > 0, zz, 0.2 * zz)
                p = jnp.exp(e - bounds_v[hh])
                plsc.store_scatter(prow_v, [b16, colh], p)

                def sbody(j, carry, hh=hh, b16=b16, p=p):
                    c0 = hh * 32 + j * 4
                    for u in range(4):
                        colv = z16i + (c0 + u)
                        hv = plsc.load_gather(hrow_v, [b16, colv])
                        plsc.store_scatter(orow_v, [b16, colv], hv * p)
                    return carry

                lax.fori_loop(0, 8, sbody, 0)
        pltpu.sync_copy(orow_v, acc_sh.at[didx_v], add=True)
        pltpu.sync_copy(prow_v, p_hbm.at[pl.ds(base, GCH)])
        return carry

    lax.fori_loop(0, EPT // GCH, chunk, 0)
    plsc.subcore_barrier()
    off = cid * NP + sid * RPT
    pltpu.sync_copy(acc_sh.at[pl.ds(sid * RPT, RPT)],
                    out_v_hbm.at[pl.ds(off, RPT)])


def _trans_edge_body(sidx_hbm, didx_hbm, ckv_hbm, dq_hbm, bounds_hbm,
                     zeros_v_hbm,
                     out_v_hbm, p_hbm,
                     sidx_v, didx_v, kvrow_v, qrow_v, orow_v, prow_v,
                     bounds_v, acc_sh, sem1, sem2):
    cid = lax.axis_index("c")
    sid = lax.axis_index("s")
    pltpu.sync_copy(zeros_v_hbm, acc_sh.at[pl.ds(sid * RPT, RPT)])
    pltpu.sync_copy(bounds_hbm, bounds_v)
    plsc.subcore_barrier()

    def chunk(g, carry):
        base = cid * EPC + sid * EPT + g * TCH
        pltpu.sync_copy(sidx_hbm.at[pl.ds(base, TCH)], sidx_v)
        pltpu.sync_copy(didx_hbm.at[pl.ds(base, TCH)], didx_v)
        cp1 = pltpu.async_copy(ckv_hbm.at[sidx_v], kvrow_v, sem1)
        cp2 = pltpu.async_copy(dq_hbm.at[didx_v], qrow_v, sem2)
        cp1.wait()
        cp2.wait()
        for sub in range(TCH // 16):
            b16 = lax.iota(jnp.int32, 16) + sub * 16
            z16i = jnp.zeros((16,), jnp.int32)
            for hh in range(H):

                def dbody(j, acc, hh=hh, b16=b16):
                    c0 = hh * 32 + j * 4
                    for u in range(4):
                        colv = z16i + (c0 + u)
                        qc = plsc.load_gather(qrow_v, [b16, colv])
                        kc = plsc.load_gather(kvrow_v, [b16, colv])
                        acc = acc + qc * kc
                    return acc

                dot_h = lax.fori_loop(0, 8, dbody, jnp.zeros((16,), jnp.float32))
                p = jnp.exp(dot_h - bounds_v[hh])
                plsc.store_scatter(prow_v, [b16, jnp.full((16,), hh, jnp.int32)], p)

                def vbody(j, carry, hh=hh, b16=b16, p=p):
                    c0 = hh * 32 + j * 4
                    for u in range(4):
                        colv = z16i + (c0 + u)
                        vv = plsc.load_gather(kvrow_v, [b16, colv + 128])
                        plsc.store_scatter(orow_v, [b16, colv], vv * p)
                    return carry

                lax.fori_loop(0, 8, vbody, 0)
        pltpu.sync_copy(orow_v, acc_sh.at[didx_v], add=True)
        pltpu.sync_copy(prow_v, p_hbm.at[pl.ds(base, TCH)])
        return carry

    lax.fori_loop(0, EPT // TCH, chunk, 0)
    plsc.subcore_barrier()
    off = cid * NP + sid * RPT
    pltpu.sync_copy(acc_sh.at[pl.ds(sid * RPT, RPT)],
                    out_v_hbm.at[pl.ds(off, RPT)])


def _denom_body(didx_hbm, pg_hbm, pt_hbm, zeros_s_hbm,
                sg_hbm, st_hbm,
                didx_v, prow_v, sacc_v, sem1):
    cid = lax.axis_index("c")
    sid = lax.axis_index("s")
    pltpu.sync_copy(zeros_s_hbm, sacc_v)
    plsc.subcore_barrier()

    def make_loop(p_hbm):
        def chunk(g, carry):
            base = sid * EPTD + g * DCH
            pltpu.sync_copy(didx_hbm.at[pl.ds(base, DCH)], didx_v)
            pltpu.sync_copy(p_hbm.at[pl.ds(base, DCH)], prow_v)
            for sub in range(DCH // 16):
                b16 = lax.iota(jnp.int32, 16) + sub * 16
                didx16 = didx_v[pl.ds(sub * 16, 16)]
                for hh in range(H):
                    pv = plsc.load_gather(prow_v, [b16, jnp.full((16,), hh, jnp.int32)])
                    flat = lax.shift_left(didx16, 2) + hh
                    ridx = lax.shift_right_logical(flat, 7)
                    lidx = jnp.bitwise_and(flat, 127)
                    plsc.addupdate_scatter(sacc_v, [ridx, lidx], pv)
            return carry
        return chunk

    @pl.when(cid == 0)
    def _():
        lax.fori_loop(0, EPTD // DCH, make_loop(pg_hbm), 0)
        pltpu.sync_copy(sacc_v, sg_hbm.at[pl.ds(sid * SROWS, SROWS)])

    @pl.when(cid == 1)
    def _():
        lax.fori_loop(0, EPTD // DCH, make_loop(pt_hbm), 0)
        pltpu.sync_copy(sacc_v, st_hbm.at[pl.ds(sid * SROWS, SROWS)])


def _gat_edge(sidx, didx, h_tab, ald_tab, bounds, zeros_v):
    f = pl.kernel(
        _gat_edge_body,
        out_type=[
            jax.ShapeDtypeStruct((2 * NP, D), jnp.float32),
            jax.ShapeDtypeStruct((EP, H), jnp.float32),
        ],
        mesh=_MESH,
        scratch_types=[
            pltpu.VMEM((GCH,), jnp.int32),
            pltpu.VMEM((GCH,), jnp.int32),
            pltpu.VMEM((GCH, 2 * D), jnp.float32),
            pltpu.VMEM((GCH, D), jnp.float32),
            pltpu.VMEM((GCH, D), jnp.float32),
            pltpu.VMEM((GCH, H), jnp.float32),
            pltpu.VMEM((8, 16), jnp.float32),
            pltpu.VMEM_SHARED((NP, D), jnp.float32),
            pltpu.SemaphoreType.DMA,
            pltpu.SemaphoreType.DMA,
        ],
        compiler_params=_SC_PARAMS,
    )
    return f(sidx, didx, h_tab, ald_tab, bounds, zeros_v)


def _trans_edge(sidx, didx, ckv_tab, dq_tab, bounds, zeros_v):
    f = pl.kernel(
        _trans_edge_body,
        out_type=[
            jax.ShapeDtypeStruct((2 * NP, D), jnp.float32),
            jax.ShapeDtypeStruct((EP, H), jnp.float32),
        ],
        mesh=_MESH,
        scratch_types=[
            pltpu.VMEM((TCH,), jnp.int32),
            pltpu.VMEM((TCH,), jnp.int32),
            pltpu.VMEM((TCH, 2 * D), jnp.float32),
            pltpu.VMEM((TCH, D), jnp.float32),
            pltpu.VMEM((TCH, D), jnp.float32),
            pltpu.VMEM((TCH, H), jnp.float32),
            pltpu.VMEM((8, 16), jnp.float32),
            pltpu.VMEM_SHARED((NP, D), jnp.float32),
            pltpu.SemaphoreType.DMA,
            pltpu.SemaphoreType.DMA,
        ],
        compiler_params=_SC_PARAMS,
    )
    return f(sidx, didx, ckv_tab, dq_tab, bounds, zeros_v)


def _denom(didx, pg, pt, zeros_s):
    f = pl.kernel(
        _denom_body,
        out_type=[
            jax.ShapeDtypeStruct((16 * SROWS, D), jnp.float32),
            jax.ShapeDtypeStruct((16 * SROWS, D), jnp.float32),
        ],
        mesh=_MESH,
        scratch_types=[
            pltpu.VMEM((DCH,), jnp.int32),
            pltpu.VMEM((DCH, H), jnp.float32),
            pltpu.VMEM((SROWS, D), jnp.float32),
            pltpu.SemaphoreType.DMA,
        ],
        compiler_params=_SC_PARAMS,
    )
    return f(didx, pg, pt, zeros_s)


def _hop_post_body(ag0_ref, ag1_ref, at0_ref, at1_ref, sg_ref, st_ref,
                   h_ref, als_ref, ald_ref, skip_ref, emask_ref, emask4_ref,
                   gbound_ref, bg_ref, out_ref):
    accg = ag0_ref[...] + ag1_ref[...]
    acct = at0_ref[...] + at1_ref[...]
    sg = jnp.sum(sg_ref[...], axis=0)
    st = jnp.sum(st_ref[...], axis=0)
    h = h_ref[...]
    em = emask_ref[...]
    em4 = emask4_ref[...]
    als = jnp.dot(als_ref[...], em, preferred_element_type=jnp.float32)
    ald = jnp.dot(ald_ref[...], em, preferred_element_type=jnp.float32)
    zz = als + ald
    e = jnp.where(zz > 0, zz, 0.2 * zz)
    p_self = jnp.exp(e - gbound_ref[...])
    sg_l = jnp.dot(sg, em4, preferred_element_type=jnp.float32)
    gat = (accg + p_self * h) / (sg_l + p_self + 1e-16) + bg_ref[...]
    st_l = jnp.dot(st, em4, preferred_element_type=jnp.float32)
    tr = acct / (st_l + 1e-16) + skip_ref[...]
    out_ref[...] = 0.5 * (gat + tr)


def _hop_post(accg_v, sg_out, acct_v, st_out, h_tab, als_tab, ald_tab, skip,
              emask, emask4, gbound_col, bg_row):
    nb = NP // 128
    sg3 = sg_out.reshape(16, NP, H)
    st3 = st_out.reshape(16, NP, H)
    return pl.pallas_call(
        _hop_post_body,
        grid=(nb,),
        in_specs=[
            pl.BlockSpec((128, D), lambda i: (i, 0)),
            pl.BlockSpec((128, D), lambda i: (i + nb, 0)),
            pl.BlockSpec((128, D), lambda i: (i, 0)),
            pl.BlockSpec((128, D), lambda i: (i + nb, 0)),
            pl.BlockSpec((16, 128, H), lambda i: (0, i, 0)),
            pl.BlockSpec((16, 128, H), lambda i: (0, i, 0)),
            pl.BlockSpec((128, D), lambda i: (i, 0)),
            pl.BlockSpec((128, 16), lambda i: (i, 0)),
            pl.BlockSpec((128, 16), lambda i: (i, 0)),
            pl.BlockSpec((128, D), lambda i: (i, 0)),
            pl.BlockSpec((16, D), lambda i: (0, 0)),
            pl.BlockSpec((H, D), lambda i: (0, 0)),
            pl.BlockSpec((1, D), lambda i: (0, 0)),
            pl.BlockSpec((1, D), lambda i: (0, 0)),
        ],
        out_specs=pl.BlockSpec((128, D), lambda i: (i, 0)),
        out_shape=jax.ShapeDtypeStruct((NP, D), jnp.float32),
    )(accg_v, accg_v, acct_v, acct_v, sg3, st3, h_tab, als_tab, ald_tab,
      skip, emask, emask4, gbound_col, bg_row)


def _epilogue_body(x_ref, h0_ref, h1_ref, h2_ref, w_ref,
                   se_w1_ref, se_b1_ref, se_w2_ref, se_b2_ref,
                   wf_ref, bf_ref, ln_g_ref, ln_b_ref, out_ref):
    xr = x_ref[0]
    wv = w_ref[...]
    wh = h0_ref[0] * wv[0:1] + h1_ref[0] * wv[1:2] + h2_ref[0] * wv[2:3]
    pool = jnp.mean(wh, axis=0, keepdims=True)
    t1 = _gelu_exact(pool @ se_w1_ref[...] + se_b1_ref[...])
    se = jax.nn.sigmoid(t1 @ se_w2_ref[...] + se_b2_ref[...])
    whr = wh * se
    f = (xr @ wf_ref[0] + whr @ wf_ref[1]) + bf_ref[...]
    mu = jnp.mean(f, axis=-1, keepdims=True)
    var = jnp.mean((f - mu) ** 2, axis=-1, keepdims=True)
    f = (f - mu) * jax.lax.rsqrt(var + 1e-5) * ln_g_ref[...] + ln_b_ref[...]
    out_ref[0] = _gelu_exact(f) + xr


def _epilogue(x, h0, h1, h2, wrow, se_w1, se_b1, se_w2, se_b2, Wf, bf, ln_g, ln_b):
    wf2 = Wf.reshape(2, D, D)
    out = pl.pallas_call(
        _epilogue_body,
        grid=(B,),
        in_specs=[
            pl.BlockSpec((1, S, D), lambda i: (i, 0, 0)),
            pl.BlockSpec((1, S, D), lambda i: (i, 0, 0)),
            pl.BlockSpec((1, S, D), lambda i: (i, 0, 0)),
            pl.BlockSpec((1, S, D), lambda i: (i, 0, 0)),
            pl.BlockSpec((3, D), lambda i: (0, 0)),
            pl.BlockSpec((D, D // 8), lambda i: (0, 0)),
            pl.BlockSpec((1, D // 8), lambda i: (0, 0)),
            pl.BlockSpec((D // 8, D), lambda i: (0, 0)),
            pl.BlockSpec((1, D), lambda i: (0, 0)),
            pl.BlockSpec((2, D, D), lambda i: (0, 0, 0)),
            pl.BlockSpec((1, D), lambda i: (0, 0)),
            pl.BlockSpec((1, D), lambda i: (0, 0)),
            pl.BlockSpec((1, D), lambda i: (0, 0)),
        ],
        out_specs=pl.BlockSpec((1, S, D), lambda i: (i, 0, 0)),
        out_shape=jax.ShapeDtypeStruct((B, S, D), jnp.float32),
    )(x.reshape(B, S, D), h0.reshape(B, S, D), h1.reshape(B, S, D),
      h2.reshape(B, S, D), wrow, se_w1, se_b1.reshape(1, -1),
      se_w2, se_b2.reshape(1, -1), wf2, bf.reshape(1, -1),
      ln_g.reshape(1, -1), ln_b.reshape(1, -1))
    return out.reshape(N, D)


def kernel(x, edge_index, Wg, att_src, att_dst, bg, Wq, bq, Wk, bk, Wv, bv,
           Ws, bs, hop_w, se_w1, se_b1, se_w2, se_b2, Wf, bf, ln_g, ln_b):
    f32 = jnp.float32
    isq = 1.0 / jnp.sqrt(jnp.asarray(C, f32))
    src = edge_index[0]
    dst = edge_index[1]
    pad_e = jnp.full((EP - E,), N, jnp.int32)
    sidx = jnp.concatenate([src.astype(jnp.int32), pad_e])
    didx = jnp.concatenate([dst.astype(jnp.int32), pad_e])

    xp = jnp.zeros((NP, D), f32).at[:N].set(x)

    ar = jnp.arange(D)
    smask = (ar[:, None] // C == jnp.arange(16)[None, :]).astype(f32)
    emask = (jnp.arange(16)[:, None] == ar[None, :] // C).astype(f32)
    emask4 = emask[:H]
    zeros_v = jnp.zeros((RPT, D), f32)
    zeros_s = jnp.zeros((SROWS, D), f32)

    hops = []
    xi = xp
    for i in range(HOPS):
        wcat = jnp.concatenate(
            [Wg[i], Wq[i] * isq, Wk[i], Wv[i], Ws[i]], axis=1)
        bcat = jnp.concatenate(
            [jnp.zeros((D,), f32), bq[i] * isq, bk[i], bv[i], bs[i]]).reshape(1, -1)
        asrc_flat = att_src[i].reshape(1, D)
        adst_flat = att_dst[i].reshape(1, D)

        h_tab, ald_tab, ckv_tab, dq_tab, skip = _hop_pre(
            xi, wcat, bcat, asrc_flat, adst_flat, smask)
        als_tab = h_tab[:, 128:144]
        ald_tab16 = ald_tab[:, 0:16]

        als4 = als_tab[:, 0:4]
        ald4 = ald_tab[:, 0:4]
        zb = als4.max(axis=0) + ald4.max(axis=0)
        gb = jnp.where(zb > 0, zb, 0.2 * zb)
        qn = jnp.sqrt((dq_tab.reshape(NP, H, C) ** 2).sum(-1)).max(axis=0)
        kn = jnp.sqrt((ckv_tab[:, :D].reshape(NP, H, C) ** 2).sum(-1)).max(axis=0)
        tb = qn * kn
        gb_sc = jnp.zeros((8, 16), f32).at[0:4].set(
            jnp.broadcast_to(gb[:, None], (4, 16)))
        tb_sc = jnp.zeros((8, 16), f32).at[0:4].set(
            jnp.broadcast_to(tb[:, None], (4, 16)))
        gb_col = gb[ar // C].reshape(1, D)

        accg_v, pg = _gat_edge(sidx, didx, h_tab, ald_tab, gb_sc, zeros_v)
        acct_v, pt = _trans_edge(sidx, didx, ckv_tab, dq_tab, tb_sc, zeros_v)
        sg_out, st_out = _denom(didx, pg, pt, zeros_s)

        xi = _hop_post(accg_v, sg_out, acct_v, st_out, h_tab[:, 0:128],
                       als_tab, ald_tab16, skip, emask, emask4, gb_col,
                       bg[i].reshape(1, D))
        hops.append(xi)

    w = jax.nn.softmax(hop_w).astype(f32)
    wrow = jnp.broadcast_to(w[:, None], (3, D))
    return _epilogue(x, hops[0][:N], hops[1][:N], hops[2][:N], wrow,
                     se_w1, se_b1, se_w2, se_b2, Wf, bf, ln_g, ln_b)


# speedup vs baseline: 41.6130x; 3.3558x over previous
import functools

import jax
import jax.numpy as jnp
from jax import lax
from jax.experimental import pallas as pl
from jax.experimental.pallas import tpu as pltpu
from jax.experimental.pallas import tpu_sc as plsc

N = 10150
E = 324800
D = 128
H = 4
C = 32
HOPS = 3
B = 50
S = 203

NP = 10240
EP = 327680
EPC = EP // 2
EPT = EP // 32
CHUNK = 128
NCHUNK = EPT // CHUNK
RPT = NP // 16
SROWS = NP * H // 128


def _gelu_exact(x):
    return 0.5 * x * (1.0 + jax.lax.erf(x * 0.7071067811865476))


def _hop_pre_body(x_ref, w_ref, b_ref, asrc_ref, adst_ref, smask_ref,
                  h_ref, ald_ref, ckv_ref, dq_ref, skip_ref):
    xb = x_ref[...]
    t = jnp.dot(xb, w_ref[...], preferred_element_type=jnp.float32) + b_ref[...]
    h = t[:, 0:128]
    q = t[:, 128:256]
    k = t[:, 256:384]
    v = t[:, 384:512]
    sk = t[:, 512:640]
    als = jnp.dot(h * asrc_ref[...], smask_ref[...],
                  preferred_element_type=jnp.float32)
    ald = jnp.dot(h * adst_ref[...], smask_ref[...],
                  preferred_element_type=jnp.float32)
    z112 = jnp.zeros((128, 112), jnp.float32)
    h_ref[...] = jnp.concatenate([h, als, z112], axis=1)
    ald_ref[...] = jnp.concatenate([ald, z112], axis=1)
    ckv_ref[...] = jnp.concatenate([k, v], axis=1)
    dq_ref[...] = q
    skip_ref[...] = sk


def _hop_pre(xp, wcat, bcat, asrc_flat, adst_flat, smask):
    grid = (NP // 128,)
    return pl.pallas_call(
        _hop_pre_body,
        grid=grid,
        in_specs=[
            pl.BlockSpec((128, D), lambda i: (i, 0)),
            pl.BlockSpec((D, 5 * D), lambda i: (0, 0)),
            pl.BlockSpec((1, 5 * D), lambda i: (0, 0)),
            pl.BlockSpec((1, D), lambda i: (0, 0)),
            pl.BlockSpec((1, D), lambda i: (0, 0)),
            pl.BlockSpec((D, 16), lambda i: (0, 0)),
        ],
        out_specs=[
            pl.BlockSpec((128, 2 * D), lambda i: (i, 0)),
            pl.BlockSpec((128, D), lambda i: (i, 0)),
            pl.BlockSpec((128, 2 * D), lambda i: (i, 0)),
            pl.BlockSpec((128, D), lambda i: (i, 0)),
            pl.BlockSpec((128, D), lambda i: (i, 0)),
        ],
        out_shape=[
            jax.ShapeDtypeStruct((NP, 2 * D), jnp.float32),
            jax.ShapeDtypeStruct((NP, D), jnp.float32),
            jax.ShapeDtypeStruct((NP, 2 * D), jnp.float32),
            jax.ShapeDtypeStruct((NP, D), jnp.float32),
            jax.ShapeDtypeStruct((NP, D), jnp.float32),
        ],
    )(xp, wcat, bcat, asrc_flat, adst_flat, smask)


_MESH = plsc.VectorSubcoreMesh(core_axis_name="c", subcore_axis_name="s")
_SC_PARAMS = pltpu.CompilerParams(needs_layout_passes=False)

GCH = 32
TCH = 32
DCH = 256
EPTD = EP // 16


def _gat_edge_body(sidx_hbm, didx_hbm, h_hbm, ald_hbm, bounds_hbm,
                   zeros_v_hbm,
                   out_v_hbm, p_hbm,
                   sidx_v, didx_v, hrow_v, adrow_v, orow_v, prow_v,
                   bounds_v, acc_sh, semh, sema):
    cid = lax.axis_index("c")
    sid = lax.axis_index("s")
    pltpu.sync_copy(zeros_v_hbm, acc_sh.at[pl.ds(sid * RPT, RPT)])
    pltpu.sync_copy(bounds_hbm, bounds_v)
    plsc.subcore_barrier()
    nchunk = EPT // GCH
    tbase = cid * EPC + sid * EPT

    pltpu.sync_copy(sidx_hbm.at[pl.ds(tbase, GCH)], sidx_v.at[0])
    pltpu.sync_copy(didx_hbm.at[pl.ds(tbase, GCH)], didx_v.at[0])
    pltpu.sync_copy(sidx_hbm.at[pl.ds(tbase + GCH, GCH)], sidx_v.at[1])
    pltpu.sync_copy(didx_hbm.at[pl.ds(tbase + GCH, GCH)], didx_v.at[1])
    pltpu.async_copy(h_hbm.at[sidx_v.at[0]], hrow_v.at[0], semh)
    pltpu.async_copy(ald_hbm.at[didx_v.at[0]], adrow_v.at[0], sema)

    def pair(gp, carry):
        g0 = gp * 2
        for b in range(2):
            g = g0 + b
            nb = 1 - b
            pltpu.make_async_copy(h_hbm.at[sidx_v.at[b]], hrow_v.at[b], semh).wait()
            pltpu.make_async_copy(ald_hbm.at[didx_v.at[b]], adrow_v.at[b], sema).wait()
            pltpu.async_copy(h_hbm.at[sidx_v.at[nb]], hrow_v.at[nb], semh)
            pltpu.async_copy(ald_hbm.at[didx_v.at[nb]], adrow_v.at[nb], sema)
            hrow_b = hrow_v.at[b]
            adrow_b = adrow_v.at[b]
            lane16 = lax.iota(jnp.int32, 16)
            blane = bounds_v[4]
            for sub in range(GCH // 16):
                pvecs = [jnp.zeros((16,), jnp.float32) for _ in range(H)]
                for i in range(16):
                    ei = sub * 16 + i
                    asv = hrow_b[ei, pl.ds(128, 16)]
                    adv = adrow_b[ei, pl.ds(0, 16)]
                    zz = asv + adv
                    ev = jnp.where(zz > 0, zz, 0.2 * zz)
                    pe = jnp.exp(ev - blane)
                    sel = lane16 == i
                    for hh in range(H):
                        sp = jnp.take_along_axis(
                            pe, jnp.full((16,), hh, jnp.int32), axis=0,
                            mode="promise_in_bounds")
                        pvecs[hh] = jnp.where(sel, sp, pvecs[hh])
                        c0 = hh * 32
                        orow_v[ei, pl.ds(c0, 16)] = hrow_b[ei, pl.ds(c0, 16)] * sp
                        orow_v[ei, pl.ds(c0 + 16, 16)] = (
                            hrow_b[ei, pl.ds(c0 + 16, 16)] * sp)
                for hh in range(H):
                    prow_v[pl.ds(hh * GCH + sub * 16, 16)] = pvecs[hh]
            base = tbase + g * GCH
            pltpu.sync_copy(orow_v, acc_sh.at[didx_v.at[b]], add=True)
            pltpu.sync_copy(prow_v, p_hbm.at[pl.ds(4 * base, 4 * GCH)])
            nxt = jnp.where(g + 2 < nchunk, g + 2, 0)
            pltpu.sync_copy(sidx_hbm.at[pl.ds(tbase + nxt * GCH, GCH)], sidx_v.at[b])
            pltpu.sync_copy(didx_hbm.at[pl.ds(tbase + nxt * GCH, GCH)], didx_v.at[b])
        return carry

    lax.fori_loop(0, nchunk // 2, pair, 0)
    pltpu.make_async_copy(h_hbm.at[sidx_v.at[0]], hrow_v.at[0], semh).wait()
    pltpu.make_async_copy(ald_hbm.at[didx_v.at[0]], adrow_v.at[0], sema).wait()
    plsc.subcore_barrier()
    off = cid * NP + sid * RPT
    pltpu.sync_copy(acc_sh.at[pl.ds(sid * RPT, RPT)],
                    out_v_hbm.at[pl.ds(off, RPT)])


def _trans_edge_body(sidx_hbm, didx_hbm, ckv_hbm, dq_hbm, bounds_hbm,
                     zeros_v_hbm,
                     out_v_hbm, p_hbm,
                     sidx_v, didx_v, kvrow_v, qrow_v, orow_v, prow_v,
                     bounds_v, acc_sh, semk, semq):
    cid = lax.axis_index("c")
    sid = lax.axis_index("s")
    pltpu.sync_copy(zeros_v_hbm, acc_sh.at[pl.ds(sid * RPT, RPT)])
    pltpu.sync_copy(bounds_hbm, bounds_v)
    plsc.subcore_barrier()
    nchunk = EPT // TCH
    tbase = cid * EPC + sid * EPT

    pltpu.sync_copy(sidx_hbm.at[pl.ds(tbase, TCH)], sidx_v.at[0])
    pltpu.sync_copy(didx_hbm.at[pl.ds(tbase, TCH)], didx_v.at[0])
    pltpu.sync_copy(sidx_hbm.at[pl.ds(tbase + TCH, TCH)], sidx_v.at[1])
    pltpu.sync_copy(didx_hbm.at[pl.ds(tbase + TCH, TCH)], didx_v.at[1])
    pltpu.async_copy(ckv_hbm.at[sidx_v.at[0]], kvrow_v.at[0], semk)
    pltpu.async_copy(dq_hbm.at[didx_v.at[0]], qrow_v.at[0], semq)

    def pair(gp, carry):
        g0 = gp * 2
        for b in range(2):
            g = g0 + b
            nb = 1 - b
            pltpu.make_async_copy(ckv_hbm.at[sidx_v.at[b]], kvrow_v.at[b], semk).wait()
            pltpu.make_async_copy(dq_hbm.at[didx_v.at[b]], qrow_v.at[b], semq).wait()
            pltpu.async_copy(ckv_hbm.at[sidx_v.at[nb]], kvrow_v.at[nb], semk)
            pltpu.async_copy(dq_hbm.at[didx_v.at[nb]], qrow_v.at[nb], semq)
            kvrow_b = kvrow_v.at[b]
            qrow_b = qrow_v.at[b]
            lane16 = lax.iota(jnp.int32, 16)
            i15 = jnp.full((16,), 15, jnp.int32)
            bvecs = [bounds_v[hh] for hh in range(H)]
            for sub in range(TCH // 16):
                pvecs = [jnp.zeros((16,), jnp.float32) for _ in range(H)]
                for i in range(16):
                    ei = sub * 16 + i
                    sel = lane16 == i
                    for hh in range(H):
                        c0 = hh * 32
                        m0 = (qrow_b[ei, pl.ds(c0, 16)]
                              * kvrow_b[ei, pl.ds(c0, 16)])
                        m1 = (qrow_b[ei, pl.ds(c0 + 16, 16)]
                              * kvrow_b[ei, pl.ds(c0 + 16, 16)])
                        cs = plsc.cumsum(m0 + m1)
                        tot = jnp.take_along_axis(cs, i15, axis=0,
                                                  mode="promise_in_bounds")
                        sp = jnp.exp(tot - bvecs[hh])
                        pvecs[hh] = jnp.where(sel, sp, pvecs[hh])
                        orow_v[ei, pl.ds(c0, 16)] = (
                            kvrow_b[ei, pl.ds(128 + c0, 16)] * sp)
                        orow_v[ei, pl.ds(c0 + 16, 16)] = (
                            kvrow_b[ei, pl.ds(144 + c0, 16)] * sp)
                for hh in range(H):
                    prow_v[pl.ds(hh * TCH + sub * 16, 16)] = pvecs[hh]
            base = tbase + g * TCH
            pltpu.sync_copy(orow_v, acc_sh.at[didx_v.at[b]], add=True)
            pltpu.sync_copy(prow_v, p_hbm.at[pl.ds(4 * base, 4 * TCH)])
            nxt = jnp.where(g + 2 < nchunk, g + 2, 0)
            pltpu.sync_copy(sidx_hbm.at[pl.ds(tbase + nxt * TCH, TCH)], sidx_v.at[b])
            pltpu.sync_copy(didx_hbm.at[pl.ds(tbase + nxt * TCH, TCH)], didx_v.at[b])
        return carry

    lax.fori_loop(0, nchunk // 2, pair, 0)
    pltpu.make_async_copy(ckv_hbm.at[sidx_v.at[0]], kvrow_v.at[0], semk).wait()
    pltpu.make_async_copy(dq_hbm.at[didx_v.at[0]], qrow_v.at[0], semq).wait()
    plsc.subcore_barrier()
    off = cid * NP + sid * RPT
    pltpu.sync_copy(acc_sh.at[pl.ds(sid * RPT, RPT)],
                    out_v_hbm.at[pl.ds(off, RPT)])


def _denom_body(didx_hbm, pg_hbm, pt_hbm, zeros_s_hbm,
                sg_hbm, st_hbm,
                didx_v, prow_v, sacc_v, sem1):
    cid = lax.axis_index("c")
    sid = lax.axis_index("s")
    pltpu.sync_copy(zeros_s_hbm, sacc_v)
    plsc.subcore_barrier()

    def make_loop(p_hbm):
        def chunk(g, carry):
            base = sid * EPTD + g * DCH
            pltpu.sync_copy(didx_hbm.at[pl.ds(base, DCH)], didx_v)
            pltpu.sync_copy(p_hbm.at[pl.ds(4 * base, 4 * DCH)], prow_v)
            for sub in range(DCH // 16):
                kb = sub // 2
                sub2 = sub % 2
                didx16 = didx_v[pl.ds(sub * 16, 16)]
                for hh in range(H):
                    pv = prow_v[pl.ds(kb * 128 + hh * 32 + sub2 * 16, 16)]
                    flat = lax.shift_left(didx16, 2) + hh
                    ridx = lax.shift_right_logical(flat, 7)
                    lidx = jnp.bitwise_and(flat, 127)
                    plsc.addupdate_scatter(sacc_v, [ridx, lidx], pv)
            return carry
        return chunk

    @pl.when(cid == 0)
    def _():
        lax.fori_loop(0, EPTD // DCH, make_loop(pg_hbm), 0)
        pltpu.sync_copy(sacc_v, sg_hbm.at[pl.ds(sid * SROWS, SROWS)])

    @pl.when(cid == 1)
    def _():
        lax.fori_loop(0, EPTD // DCH, make_loop(pt_hbm), 0)
        pltpu.sync_copy(sacc_v, st_hbm.at[pl.ds(sid * SROWS, SROWS)])


def _gat_edge(sidx, didx, h_tab, ald_tab, bounds, zeros_v):
    f = pl.kernel(
        _gat_edge_body,
        out_type=[
            jax.ShapeDtypeStruct((2 * NP, D), jnp.float32),
            jax.ShapeDtypeStruct((4 * EP,), jnp.float32),
        ],
        mesh=_MESH,
        scratch_types=[
            pltpu.VMEM((2, GCH), jnp.int32),
            pltpu.VMEM((2, GCH), jnp.int32),
            pltpu.VMEM((2, GCH, 2 * D), jnp.float32),
            pltpu.VMEM((2, GCH, D), jnp.float32),
            pltpu.VMEM((GCH, D), jnp.float32),
            pltpu.VMEM((4 * GCH,), jnp.float32),
            pltpu.VMEM((8, 16), jnp.float32),
            pltpu.VMEM_SHARED((NP, D), jnp.float32),
            pltpu.SemaphoreType.DMA,
            pltpu.SemaphoreType.DMA,
        ],
        compiler_params=_SC_PARAMS,
    )
    return f(sidx, didx, h_tab, ald_tab, bounds, zeros_v)


def _trans_edge(sidx, didx, ckv_tab, dq_tab, bounds, zeros_v):
    f = pl.kernel(
        _trans_edge_body,
        out_type=[
            jax.ShapeDtypeStruct((2 * NP, D), jnp.float32),
            jax.ShapeDtypeStruct((4 * EP,), jnp.float32),
        ],
        mesh=_MESH,
        scratch_types=[
            pltpu.VMEM((2, TCH), jnp.int32),
            pltpu.VMEM((2, TCH), jnp.int32),
            pltpu.VMEM((2, TCH, 2 * D), jnp.float32),
            pltpu.VMEM((2, TCH, D), jnp.float32),
            pltpu.VMEM((TCH, D), jnp.float32),
            pltpu.VMEM((4 * TCH,), jnp.float32),
            pltpu.VMEM((8, 16), jnp.float32),
            pltpu.VMEM_SHARED((NP, D), jnp.float32),
            pltpu.SemaphoreType.DMA,
            pltpu.SemaphoreType.DMA,
        ],
        compiler_params=_SC_PARAMS,
    )
    return f(sidx, didx, ckv_tab, dq_tab, bounds, zeros_v)


def _denom(didx, pg, pt, zeros_s):
    f = pl.kernel(
        _denom_body,
        out_type=[
            jax.ShapeDtypeStruct((16 * SROWS, D), jnp.float32),
            jax.ShapeDtypeStruct((16 * SROWS, D), jnp.float32),
        ],
        mesh=_MESH,
        scratch_types=[
            pltpu.VMEM((DCH,), jnp.int32),
            pltpu.VMEM((4 * DCH,), jnp.float32),
            pltpu.VMEM((SROWS, D), jnp.float32),
            pltpu.SemaphoreType.DMA,
        ],
        compiler_params=_SC_PARAMS,
    )
    return f(didx, pg, pt, zeros_s)


def _hop_post_body(ag0_ref, ag1_ref, at0_ref, at1_ref, sg_ref, st_ref,
                   h_ref, als_ref, ald_ref, skip_ref, emask_ref, emask4_ref,
                   gbound_ref, bg_ref, out_ref):
    accg = ag0_ref[...] + ag1_ref[...]
    acct = at0_ref[...] + at1_ref[...]
    sg = jnp.sum(sg_ref[...], axis=0)
    st = jnp.sum(st_ref[...], axis=0)
    h = h_ref[...]
    em = emask_ref[...]
    em4 = emask4_ref[...]
    als = jnp.dot(als_ref[...], em, preferred_element_type=jnp.float32)
    ald = jnp.dot(ald_ref[...], em, preferred_element_type=jnp.float32)
    zz = als + ald
    e = jnp.where(zz > 0, zz, 0.2 * zz)
    p_self = jnp.exp(e - gbound_ref[...])
    sg_l = jnp.dot(sg, em4, preferred_element_type=jnp.float32)
    gat = (accg + p_self * h) / (sg_l + p_self + 1e-16) + bg_ref[...]
    st_l = jnp.dot(st, em4, preferred_element_type=jnp.float32)
    tr = acct / (st_l + 1e-16) + skip_ref[...]
    out_ref[...] = 0.5 * (gat + tr)


def _hop_post(accg_v, sg_out, acct_v, st_out, h_tab, als_tab, ald_tab, skip,
              emask, emask4, gbound_col, bg_row):
    nb = NP // 128
    sg3 = sg_out.reshape(16, NP, H)
    st3 = st_out.reshape(16, NP, H)
    return pl.pallas_call(
        _hop_post_body,
        grid=(nb,),
        in_specs=[
            pl.BlockSpec((128, D), lambda i: (i, 0)),
            pl.BlockSpec((128, D), lambda i: (i + nb, 0)),
            pl.BlockSpec((128, D), lambda i: (i, 0)),
            pl.BlockSpec((128, D), lambda i: (i + nb, 0)),
            pl.BlockSpec((16, 128, H), lambda i: (0, i, 0)),
            pl.BlockSpec((16, 128, H), lambda i: (0, i, 0)),
            pl.BlockSpec((128, D), lambda i: (i, 0)),
            pl.BlockSpec((128, 16), lambda i: (i, 0)),
            pl.BlockSpec((128, 16), lambda i: (i, 0)),
            pl.BlockSpec((128, D), lambda i: (i, 0)),
            pl.BlockSpec((16, D), lambda i: (0, 0)),
            pl.BlockSpec((H, D), lambda i: (0, 0)),
            pl.BlockSpec((1, D), lambda i: (0, 0)),
            pl.BlockSpec((1, D), lambda i: (0, 0)),
        ],
        out_specs=pl.BlockSpec((128, D), lambda i: (i, 0)),
        out_shape=jax.ShapeDtypeStruct((NP, D), jnp.float32),
    )(accg_v, accg_v, acct_v, acct_v, sg3, st3, h_tab, als_tab, ald_tab,
      skip, emask, emask4, gbound_col, bg_row)


def _epilogue_body(x_ref, h0_ref, h1_ref, h2_ref, w_ref,
                   se_w1_ref, se_b1_ref, se_w2_ref, se_b2_ref,
                   wf_ref, bf_ref, ln_g_ref, ln_b_ref, out_ref):
    xr = x_ref[0]
    wv = w_ref[...]
    wh = h0_ref[0] * wv[0:1] + h1_ref[0] * wv[1:2] + h2_ref[0] * wv[2:3]
    pool = jnp.mean(wh, axis=0, keepdims=True)
    t1 = _gelu_exact(pool @ se_w1_ref[...] + se_b1_ref[...])
    se = jax.nn.sigmoid(t1 @ se_w2_ref[...] + se_b2_ref[...])
    whr = wh * se
    f = (xr @ wf_ref[0] + whr @ wf_ref[1]) + bf_ref[...]
    mu = jnp.mean(f, axis=-1, keepdims=True)
    var = jnp.mean((f - mu) ** 2, axis=-1, keepdims=True)
    f = (f - mu) * jax.lax.rsqrt(var + 1e-5) * ln_g_ref[...] + ln_b_ref[...]
    out_ref[0] = _gelu_exact(f) + xr


def _epilogue(x, h0, h1, h2, wrow, se_w1, se_b1, se_w2, se_b2, Wf, bf, ln_g, ln_b):
    wf2 = Wf.reshape(2, D, D)
    out = pl.pallas_call(
        _epilogue_body,
        grid=(B,),
        in_specs=[
            pl.BlockSpec((1, S, D), lambda i: (i, 0, 0)),
            pl.BlockSpec((1, S, D), lambda i: (i, 0, 0)),
            pl.BlockSpec((1, S, D), lambda i: (i, 0, 0)),
            pl.BlockSpec((1, S, D), lambda i: (i, 0, 0)),
            pl.BlockSpec((3, D), lambda i: (0, 0)),
            pl.BlockSpec((D, D // 8), lambda i: (0, 0)),
            pl.BlockSpec((1, D // 8), lambda i: (0, 0)),
            pl.BlockSpec((D // 8, D), lambda i: (0, 0)),
            pl.BlockSpec((1, D), lambda i: (0, 0)),
            pl.BlockSpec((2, D, D), lambda i: (0, 0, 0)),
            pl.BlockSpec((1, D), lambda i: (0, 0)),
            pl.BlockSpec((1, D), lambda i: (0, 0)),
            pl.BlockSpec((1, D), lambda i: (0, 0)),
        ],
        out_specs=pl.BlockSpec((1, S, D), lambda i: (i, 0, 0)),
        out_shape=jax.ShapeDtypeStruct((B, S, D), jnp.float32),
    )(x.reshape(B, S, D), h0.reshape(B, S, D), h1.reshape(B, S, D),
      h2.reshape(B, S, D), wrow, se_w1, se_b1.reshape(1, -1),
      se_w2, se_b2.reshape(1, -1), wf2, bf.reshape(1, -1),
      ln_g.reshape(1, -1), ln_b.reshape(1, -1))
    return out.reshape(N, D)


def kernel(x, edge_index, Wg, att_src, att_dst, bg, Wq, bq, Wk, bk, Wv, bv,
           Ws, bs, hop_w, se_w1, se_b1, se_w2, se_b2, Wf, bf, ln_g, ln_b):
    f32 = jnp.float32
    isq = 1.0 / jnp.sqrt(jnp.asarray(C, f32))
    src = edge_index[0]
    dst = edge_index[1]
    pad_e = jnp.full((EP - E,), N, jnp.int32)
    sidx = jnp.concatenate([src.astype(jnp.int32), pad_e])
    didx = jnp.concatenate([dst.astype(jnp.int32), pad_e])

    xp = jnp.zeros((NP, D), f32).at[:N].set(x)

    ar = jnp.arange(D)
    smask = (ar[:, None] // C == jnp.arange(16)[None, :]).astype(f32)
    emask = (jnp.arange(16)[:, None] == ar[None, :] // C).astype(f32)
    emask4 = emask[:H]
    zeros_v = jnp.zeros((RPT, D), f32)
    zeros_s = jnp.zeros((SROWS, D), f32)

    hops = []
    xi = xp
    for i in range(HOPS):
        wcat = jnp.concatenate(
            [Wg[i], Wq[i] * isq, Wk[i], Wv[i], Ws[i]], axis=1)
        bcat = jnp.concatenate(
            [jnp.zeros((D,), f32), bq[i] * isq, bk[i], bv[i], bs[i]]).reshape(1, -1)
        asrc_flat = att_src[i].reshape(1, D)
        adst_flat = att_dst[i].reshape(1, D)

        h_tab, ald_tab, ckv_tab, dq_tab, skip = _hop_pre(
            xi, wcat, bcat, asrc_flat, adst_flat, smask)
        als_tab = h_tab[:, 128:144]
        ald_tab16 = ald_tab[:, 0:16]

        als4 = als_tab[:, 0:4]
        ald4 = ald_tab[:, 0:4]
        zb = als4.max(axis=0) + ald4.max(axis=0)
        gb = jnp.where(zb > 0, zb, 0.2 * zb)
        qn = jnp.sqrt((dq_tab.reshape(NP, H, C) ** 2).sum(-1)).max(axis=0)
        kn = jnp.sqrt((ckv_tab[:, :D].reshape(NP, H, C) ** 2).sum(-1)).max(axis=0)
        tb = qn * kn
        gb_sc = jnp.zeros((8, 16), f32).at[0:4].set(
            jnp.broadcast_to(gb[:, None], (4, 16)))
        gb_sc = gb_sc.at[4, 0:4].set(gb)
        tb_sc = jnp.zeros((8, 16), f32).at[0:4].set(
            jnp.broadcast_to(tb[:, None], (4, 16)))
        gb_col = gb[ar // C].reshape(1, D)

        accg_v, pg = _gat_edge(sidx, didx, h_tab, ald_tab, gb_sc, zeros_v)
        acct_v, pt = _trans_edge(sidx, didx, ckv_tab, dq_tab, tb_sc, zeros_v)
        sg_out, st_out = _denom(didx, pg, pt, zeros_s)

        xi = _hop_post(accg_v, sg_out, acct_v, st_out, h_tab[:, 0:128],
                       als_tab, ald_tab16, skip, emask, emask4, gb_col,
                       bg[i].reshape(1, D))
        hops.append(xi)

    w = jax.nn.softmax(hop_w).astype(f32)
    wrow = jnp.broadcast_to(w[:, None], (3, D))
    return _epilogue(x, hops[0][:N], hops[1][:N], hops[2][:N], wrow,
                     se_w1, se_b1, se_w2, se_b2, Wf, bf, ln_g, ln_b)


# speedup vs baseline: 46.3998x; 1.1150x over previous
import functools

import jax
import jax.numpy as jnp
from jax import lax
from jax.experimental import pallas as pl
from jax.experimental.pallas import tpu as pltpu
from jax.experimental.pallas import tpu_sc as plsc

N = 10150
E = 324800
D = 128
H = 4
C = 32
HOPS = 3
B = 50
S = 203

NP = 10240
EP = 327680
EPC = EP // 2
EPT = EP // 32
CHUNK = 128
NCHUNK = EPT // CHUNK
RPT = NP // 16
SROWS = NP * H // 128


def _gelu_exact(x):
    return 0.5 * x * (1.0 + jax.lax.erf(x * 0.7071067811865476))


def _hop_pre_body(x_ref, w_ref, b_ref, asrc_ref, adst_ref, smask_ref,
                  h_ref, ald_ref, ckv_ref, dq_ref, skip_ref):
    xb = x_ref[...]
    t = jnp.dot(xb, w_ref[...], preferred_element_type=jnp.float32) + b_ref[...]
    h = t[:, 0:128]
    q = t[:, 128:256]
    k = t[:, 256:384]
    v = t[:, 384:512]
    sk = t[:, 512:640]
    als = jnp.dot(h * asrc_ref[...], smask_ref[...],
                  preferred_element_type=jnp.float32)
    ald = jnp.dot(h * adst_ref[...], smask_ref[...],
                  preferred_element_type=jnp.float32)
    z112 = jnp.zeros((128, 112), jnp.float32)
    h_ref[...] = jnp.concatenate([h, als, z112], axis=1)
    ald_ref[...] = jnp.concatenate([ald, z112], axis=1)
    ckv_ref[...] = jnp.concatenate([k, v], axis=1)
    dq_ref[...] = q
    skip_ref[...] = sk


def _hop_pre(xp, wcat, bcat, asrc_flat, adst_flat, smask):
    grid = (NP // 128,)
    return pl.pallas_call(
        _hop_pre_body,
        grid=grid,
        in_specs=[
            pl.BlockSpec((128, D), lambda i: (i, 0)),
            pl.BlockSpec((D, 5 * D), lambda i: (0, 0)),
            pl.BlockSpec((1, 5 * D), lambda i: (0, 0)),
            pl.BlockSpec((1, D), lambda i: (0, 0)),
            pl.BlockSpec((1, D), lambda i: (0, 0)),
            pl.BlockSpec((D, 16), lambda i: (0, 0)),
        ],
        out_specs=[
            pl.BlockSpec((128, 2 * D), lambda i: (i, 0)),
            pl.BlockSpec((128, D), lambda i: (i, 0)),
            pl.BlockSpec((128, 2 * D), lambda i: (i, 0)),
            pl.BlockSpec((128, D), lambda i: (i, 0)),
            pl.BlockSpec((128, D), lambda i: (i, 0)),
        ],
        out_shape=[
            jax.ShapeDtypeStruct((NP, 2 * D), jnp.float32),
            jax.ShapeDtypeStruct((NP, D), jnp.float32),
            jax.ShapeDtypeStruct((NP, 2 * D), jnp.float32),
            jax.ShapeDtypeStruct((NP, D), jnp.float32),
            jax.ShapeDtypeStruct((NP, D), jnp.float32),
        ],
    )(xp, wcat, bcat, asrc_flat, adst_flat, smask)


_MESH = plsc.VectorSubcoreMesh(core_axis_name="c", subcore_axis_name="s")
_SC_PARAMS = pltpu.CompilerParams(needs_layout_passes=False)

GCH = 32
TCH = 32
DCH = 256
EPTD = EP // 16


def _gat_edge_body(sidx_hbm, didx_hbm, h_hbm, ald_hbm, bounds_hbm,
                   zeros_v_hbm,
                   out_v_hbm, p_hbm,
                   sidx_v, didx_v, hrow_v, adrow_v, orow_v, prow_v,
                   bounds_v, acc_sh, semh, sema, semi):
    cid = lax.axis_index("c")
    sid = lax.axis_index("s")
    pltpu.sync_copy(zeros_v_hbm, acc_sh.at[pl.ds(sid * RPT, RPT)])
    pltpu.sync_copy(bounds_hbm, bounds_v)
    plsc.subcore_barrier()
    nchunk = EPT // GCH
    tbase = cid * EPC + sid * EPT

    pltpu.sync_copy(sidx_hbm.at[pl.ds(tbase, GCH)], sidx_v.at[0])
    pltpu.sync_copy(didx_hbm.at[pl.ds(tbase, GCH)], didx_v.at[0])
    pltpu.async_copy(sidx_hbm.at[pl.ds(tbase + GCH, GCH)], sidx_v.at[1], semi)
    pltpu.async_copy(didx_hbm.at[pl.ds(tbase + GCH, GCH)], didx_v.at[1], semi)
    pltpu.async_copy(h_hbm.at[sidx_v.at[0]], hrow_v.at[0], semh)
    pltpu.async_copy(ald_hbm.at[didx_v.at[0]], adrow_v.at[0], sema)

    def pair(gp, carry):
        g0 = gp * 2
        for b in range(2):
            g = g0 + b
            nb = 1 - b
            pltpu.make_async_copy(h_hbm.at[sidx_v.at[b]], hrow_v.at[b], semh).wait()
            pltpu.make_async_copy(ald_hbm.at[didx_v.at[b]], adrow_v.at[b], sema).wait()
            pltpu.make_async_copy(sidx_hbm.at[pl.ds(tbase, GCH)], sidx_v.at[nb], semi).wait()
            pltpu.make_async_copy(didx_hbm.at[pl.ds(tbase, GCH)], didx_v.at[nb], semi).wait()
            pltpu.async_copy(h_hbm.at[sidx_v.at[nb]], hrow_v.at[nb], semh)
            pltpu.async_copy(ald_hbm.at[didx_v.at[nb]], adrow_v.at[nb], sema)
            hrow_b = hrow_v.at[b]
            adrow_b = adrow_v.at[b]
            lane16 = lax.iota(jnp.int32, 16)
            blane = bounds_v[4]
            for sub in range(GCH // 16):
                pvecs = [jnp.zeros((16,), jnp.float32) for _ in range(H)]
                for i in range(16):
                    ei = sub * 16 + i
                    asv = hrow_b[ei, pl.ds(128, 16)]
                    adv = adrow_b[ei, pl.ds(0, 16)]
                    zz = asv + adv
                    ev = jnp.where(zz > 0, zz, 0.2 * zz)
                    pe = jnp.exp(ev - blane)
                    sel = lane16 == i
                    for hh in range(H):
                        sp = jnp.take_along_axis(
                            pe, jnp.full((16,), hh, jnp.int32), axis=0,
                            mode="promise_in_bounds")
                        pvecs[hh] = jnp.where(sel, sp, pvecs[hh])
                        c0 = hh * 32
                        orow_v[ei, pl.ds(c0, 16)] = hrow_b[ei, pl.ds(c0, 16)] * sp
                        orow_v[ei, pl.ds(c0 + 16, 16)] = (
                            hrow_b[ei, pl.ds(c0 + 16, 16)] * sp)
                for hh in range(H):
                    prow_v[pl.ds(hh * GCH + sub * 16, 16)] = pvecs[hh]
            base = tbase + g * GCH
            pltpu.sync_copy(orow_v, acc_sh.at[didx_v.at[b]], add=True)
            pltpu.sync_copy(prow_v, p_hbm.at[pl.ds(4 * base, 4 * GCH)])
            nxt = jnp.where(g + 2 < nchunk, g + 2, 0)
            pltpu.async_copy(sidx_hbm.at[pl.ds(tbase + nxt * GCH, GCH)],
                             sidx_v.at[b], semi)
            pltpu.async_copy(didx_hbm.at[pl.ds(tbase + nxt * GCH, GCH)],
                             didx_v.at[b], semi)
        return carry

    lax.fori_loop(0, nchunk // 2, pair, 0)
    pltpu.make_async_copy(h_hbm.at[sidx_v.at[0]], hrow_v.at[0], semh).wait()
    pltpu.make_async_copy(ald_hbm.at[didx_v.at[0]], adrow_v.at[0], sema).wait()
    pltpu.make_async_copy(sidx_hbm.at[pl.ds(tbase, GCH)], sidx_v.at[1], semi).wait()
    pltpu.make_async_copy(didx_hbm.at[pl.ds(tbase, GCH)], didx_v.at[1], semi).wait()
    plsc.subcore_barrier()
    off = cid * NP + sid * RPT
    pltpu.sync_copy(acc_sh.at[pl.ds(sid * RPT, RPT)],
                    out_v_hbm.at[pl.ds(off, RPT)])


def _trans_edge_body(sidx_hbm, didx_hbm, ckv_hbm, dq_hbm, bounds_hbm,
                     zeros_v_hbm,
                     out_v_hbm, p_hbm,
                     sidx_v, didx_v, kvrow_v, qrow_v, orow_v, prow_v,
                     bounds_v, acc_sh, semk, semq, semi):
    cid = lax.axis_index("c")
    sid = lax.axis_index("s")
    pltpu.sync_copy(zeros_v_hbm, acc_sh.at[pl.ds(sid * RPT, RPT)])
    pltpu.sync_copy(bounds_hbm, bounds_v)
    plsc.subcore_barrier()
    nchunk = EPT // TCH
    tbase = cid * EPC + sid * EPT

    pltpu.sync_copy(sidx_hbm.at[pl.ds(tbase, TCH)], sidx_v.at[0])
    pltpu.sync_copy(didx_hbm.at[pl.ds(tbase, TCH)], didx_v.at[0])
    pltpu.async_copy(sidx_hbm.at[pl.ds(tbase + TCH, TCH)], sidx_v.at[1], semi)
    pltpu.async_copy(didx_hbm.at[pl.ds(tbase + TCH, TCH)], didx_v.at[1], semi)
    pltpu.async_copy(ckv_hbm.at[sidx_v.at[0]], kvrow_v.at[0], semk)
    pltpu.async_copy(dq_hbm.at[didx_v.at[0]], qrow_v.at[0], semq)

    def pair(gp, carry):
        g0 = gp * 2
        for b in range(2):
            g = g0 + b
            nb = 1 - b
            pltpu.make_async_copy(ckv_hbm.at[sidx_v.at[b]], kvrow_v.at[b], semk).wait()
            pltpu.make_async_copy(dq_hbm.at[didx_v.at[b]], qrow_v.at[b], semq).wait()
            pltpu.make_async_copy(sidx_hbm.at[pl.ds(tbase, TCH)], sidx_v.at[nb], semi).wait()
            pltpu.make_async_copy(didx_hbm.at[pl.ds(tbase, TCH)], didx_v.at[nb], semi).wait()
            pltpu.async_copy(ckv_hbm.at[sidx_v.at[nb]], kvrow_v.at[nb], semk)
            pltpu.async_copy(dq_hbm.at[didx_v.at[nb]], qrow_v.at[nb], semq)
            kvrow_b = kvrow_v.at[b]
            qrow_b = qrow_v.at[b]
            lane16 = lax.iota(jnp.int32, 16)
            i15 = jnp.full((16,), 15, jnp.int32)
            bvecs = [bounds_v[hh] for hh in range(H)]
            for sub in range(TCH // 16):
                pvecs = [jnp.zeros((16,), jnp.float32) for _ in range(H)]
                for i in range(16):
                    ei = sub * 16 + i
                    sel = lane16 == i
                    for hh in range(H):
                        c0 = hh * 32
                        m0 = (qrow_b[ei, pl.ds(c0, 16)]
                              * kvrow_b[ei, pl.ds(c0, 16)])
                        m1 = (qrow_b[ei, pl.ds(c0 + 16, 16)]
                              * kvrow_b[ei, pl.ds(c0 + 16, 16)])
                        cs = plsc.cumsum(m0 + m1)
                        tot = jnp.take_along_axis(cs, i15, axis=0,
                                                  mode="promise_in_bounds")
                        sp = jnp.exp(tot - bvecs[hh])
                        pvecs[hh] = jnp.where(sel, sp, pvecs[hh])
                        orow_v[ei, pl.ds(c0, 16)] = (
                            kvrow_b[ei, pl.ds(128 + c0, 16)] * sp)
                        orow_v[ei, pl.ds(c0 + 16, 16)] = (
                            kvrow_b[ei, pl.ds(144 + c0, 16)] * sp)
                for hh in range(H):
                    prow_v[pl.ds(hh * TCH + sub * 16, 16)] = pvecs[hh]
            base = tbase + g * TCH
            pltpu.sync_copy(orow_v, acc_sh.at[didx_v.at[b]], add=True)
            pltpu.sync_copy(prow_v, p_hbm.at[pl.ds(4 * base, 4 * TCH)])
            nxt = jnp.where(g + 2 < nchunk, g + 2, 0)
            pltpu.async_copy(sidx_hbm.at[pl.ds(tbase + nxt * TCH, TCH)],
                             sidx_v.at[b], semi)
            pltpu.async_copy(didx_hbm.at[pl.ds(tbase + nxt * TCH, TCH)],
                             didx_v.at[b], semi)
        return carry

    lax.fori_loop(0, nchunk // 2, pair, 0)
    pltpu.make_async_copy(ckv_hbm.at[sidx_v.at[0]], kvrow_v.at[0], semk).wait()
    pltpu.make_async_copy(dq_hbm.at[didx_v.at[0]], qrow_v.at[0], semq).wait()
    pltpu.make_async_copy(sidx_hbm.at[pl.ds(tbase, TCH)], sidx_v.at[1], semi).wait()
    pltpu.make_async_copy(didx_hbm.at[pl.ds(tbase, TCH)], didx_v.at[1], semi).wait()
    plsc.subcore_barrier()
    off = cid * NP + sid * RPT
    pltpu.sync_copy(acc_sh.at[pl.ds(sid * RPT, RPT)],
                    out_v_hbm.at[pl.ds(off, RPT)])


def _denom_body(didx_hbm, pg_hbm, pt_hbm, zeros_s_hbm,
                sg_hbm, st_hbm,
                didx_v, prow_v, sacc_v, sem1):
    cid = lax.axis_index("c")
    sid = lax.axis_index("s")
    pltpu.sync_copy(zeros_s_hbm, sacc_v)
    plsc.subcore_barrier()

    def make_loop(p_hbm):
        def chunk(g, carry):
            base = sid * EPTD + g * DCH
            pltpu.sync_copy(didx_hbm.at[pl.ds(base, DCH)], didx_v)
            pltpu.sync_copy(p_hbm.at[pl.ds(4 * base, 4 * DCH)], prow_v)
            for sub in range(DCH // 16):
                kb = sub // 2
                sub2 = sub % 2
                didx16 = didx_v[pl.ds(sub * 16, 16)]
                for hh in range(H):
                    pv = prow_v[pl.ds(kb * 128 + hh * 32 + sub2 * 16, 16)]
                    flat = lax.shift_left(didx16, 2) + hh
                    ridx = lax.shift_right_logical(flat, 7)
                    lidx = jnp.bitwise_and(flat, 127)
                    plsc.addupdate_scatter(sacc_v, [ridx, lidx], pv)
            return carry
        return chunk

    @pl.when(cid == 0)
    def _():
        lax.fori_loop(0, EPTD // DCH, make_loop(pg_hbm), 0)
        pltpu.sync_copy(sacc_v, sg_hbm.at[pl.ds(sid * SROWS, SROWS)])

    @pl.when(cid == 1)
    def _():
        lax.fori_loop(0, EPTD // DCH, make_loop(pt_hbm), 0)
        pltpu.sync_copy(sacc_v, st_hbm.at[pl.ds(sid * SROWS, SROWS)])


def _gat_edge(sidx, didx, h_tab, ald_tab, bounds, zeros_v):
    f = pl.kernel(
        _gat_edge_body,
        out_type=[
            jax.ShapeDtypeStruct((2 * NP, D), jnp.float32),
            jax.ShapeDtypeStruct((4 * EP,), jnp.float32),
        ],
        mesh=_MESH,
        scratch_types=[
            pltpu.VMEM((2, GCH), jnp.int32),
            pltpu.VMEM((2, GCH), jnp.int32),
            pltpu.VMEM((2, GCH, 2 * D), jnp.float32),
            pltpu.VMEM((2, GCH, D), jnp.float32),
            pltpu.VMEM((GCH, D), jnp.float32),
            pltpu.VMEM((4 * GCH,), jnp.float32),
            pltpu.VMEM((8, 16), jnp.float32),
            pltpu.VMEM_SHARED((NP, D), jnp.float32),
            pltpu.SemaphoreType.DMA,
            pltpu.SemaphoreType.DMA,
            pltpu.SemaphoreType.DMA,
        ],
        compiler_params=_SC_PARAMS,
    )
    return f(sidx, didx, h_tab, ald_tab, bounds, zeros_v)


def _trans_edge(sidx, didx, ckv_tab, dq_tab, bounds, zeros_v):
    f = pl.kernel(
        _trans_edge_body,
        out_type=[
            jax.ShapeDtypeStruct((2 * NP, D), jnp.float32),
            jax.ShapeDtypeStruct((4 * EP,), jnp.float32),
        ],
        mesh=_MESH,
        scratch_types=[
            pltpu.VMEM((2, TCH), jnp.int32),
            pltpu.VMEM((2, TCH), jnp.int32),
            pltpu.VMEM((2, TCH, 2 * D), jnp.float32),
            pltpu.VMEM((2, TCH, D), jnp.float32),
            pltpu.VMEM((TCH, D), jnp.float32),
            pltpu.VMEM((4 * TCH,), jnp.float32),
            pltpu.VMEM((8, 16), jnp.float32),
            pltpu.VMEM_SHARED((NP, D), jnp.float32),
            pltpu.SemaphoreType.DMA,
            pltpu.SemaphoreType.DMA,
            pltpu.SemaphoreType.DMA,
        ],
        compiler_params=_SC_PARAMS,
    )
    return f(sidx, didx, ckv_tab, dq_tab, bounds, zeros_v)


def _denom(didx, pg, pt, zeros_s):
    f = pl.kernel(
        _denom_body,
        out_type=[
            jax.ShapeDtypeStruct((16 * SROWS, D), jnp.float32),
            jax.ShapeDtypeStruct((16 * SROWS, D), jnp.float32),
        ],
        mesh=_MESH,
        scratch_types=[
            pltpu.VMEM((DCH,), jnp.int32),
            pltpu.VMEM((4 * DCH,), jnp.float32),
            pltpu.VMEM((SROWS, D), jnp.float32),
            pltpu.SemaphoreType.DMA,
        ],
        compiler_params=_SC_PARAMS,
    )
    return f(didx, pg, pt, zeros_s)


def _hop_post_body(ag0_ref, ag1_ref, at0_ref, at1_ref, sg_ref, st_ref,
                   h_ref, als_ref, ald_ref, skip_ref, emask_ref, emask4_ref,
                   gbound_ref, bg_ref, out_ref):
    accg = ag0_ref[...] + ag1_ref[...]
    acct = at0_ref[...] + at1_ref[...]
    sg = jnp.sum(sg_ref[...], axis=0)
    st = jnp.sum(st_ref[...], axis=0)
    h = h_ref[...]
    em = emask_ref[...]
    em4 = emask4_ref[...]
    als = jnp.dot(als_ref[...], em, preferred_element_type=jnp.float32)
    ald = jnp.dot(ald_ref[...], em, preferred_element_type=jnp.float32)
    zz = als + ald
    e = jnp.where(zz > 0, zz, 0.2 * zz)
    p_self = jnp.exp(e - gbound_ref[...])
    sg_l = jnp.dot(sg, em4, preferred_element_type=jnp.float32)
    gat = (accg + p_self * h) / (sg_l + p_self + 1e-16) + bg_ref[...]
    st_l = jnp.dot(st, em4, preferred_element_type=jnp.float32)
    tr = acct / (st_l + 1e-16) + skip_ref[...]
    out_ref[...] = 0.5 * (gat + tr)


def _hop_post(accg_v, sg_out, acct_v, st_out, h_tab, als_tab, ald_tab, skip,
              emask, emask4, gbound_col, bg_row):
    nb = NP // 128
    sg3 = sg_out.reshape(16, NP, H)
    st3 = st_out.reshape(16, NP, H)
    return pl.pallas_call(
        _hop_post_body,
        grid=(nb,),
        in_specs=[
            pl.BlockSpec((128, D), lambda i: (i, 0)),
            pl.BlockSpec((128, D), lambda i: (i + nb, 0)),
            pl.BlockSpec((128, D), lambda i: (i, 0)),
            pl.BlockSpec((128, D), lambda i: (i + nb, 0)),
            pl.BlockSpec((16, 128, H), lambda i: (0, i, 0)),
            pl.BlockSpec((16, 128, H), lambda i: (0, i, 0)),
            pl.BlockSpec((128, D), lambda i: (i, 0)),
            pl.BlockSpec((128, 16), lambda i: (i, 0)),
            pl.BlockSpec((128, 16), lambda i: (i, 0)),
            pl.BlockSpec((128, D), lambda i: (i, 0)),
            pl.BlockSpec((16, D), lambda i: (0, 0)),
            pl.BlockSpec((H, D), lambda i: (0, 0)),
            pl.BlockSpec((1, D), lambda i: (0, 0)),
            pl.BlockSpec((1, D), lambda i: (0, 0)),
        ],
        out_specs=pl.BlockSpec((128, D), lambda i: (i, 0)),
        out_shape=jax.ShapeDtypeStruct((NP, D), jnp.float32),
    )(accg_v, accg_v, acct_v, acct_v, sg3, st3, h_tab, als_tab, ald_tab,
      skip, emask, emask4, gbound_col, bg_row)


def _epilogue_body(x_ref, h0_ref, h1_ref, h2_ref, w_ref,
                   se_w1_ref, se_b1_ref, se_w2_ref, se_b2_ref,
                   wf_ref, bf_ref, ln_g_ref, ln_b_ref, out_ref):
    xr = x_ref[0]
    wv = w_ref[...]
    wh = h0_ref[0] * wv[0:1] + h1_ref[0] * wv[1:2] + h2_ref[0] * wv[2:3]
    pool = jnp.mean(wh, axis=0, keepdims=True)
    t1 = _gelu_exact(pool @ se_w1_ref[...] + se_b1_ref[...])
    se = jax.nn.sigmoid(t1 @ se_w2_ref[...] + se_b2_ref[...])
    whr = wh * se
    f = (xr @ wf_ref[0] + whr @ wf_ref[1]) + bf_ref[...]
    mu = jnp.mean(f, axis=-1, keepdims=True)
    var = jnp.mean((f - mu) ** 2, axis=-1, keepdims=True)
    f = (f - mu) * jax.lax.rsqrt(var + 1e-5) * ln_g_ref[...] + ln_b_ref[...]
    out_ref[0] = _gelu_exact(f) + xr


def _epilogue(x, h0, h1, h2, wrow, se_w1, se_b1, se_w2, se_b2, Wf, bf, ln_g, ln_b):
    wf2 = Wf.reshape(2, D, D)
    out = pl.pallas_call(
        _epilogue_body,
        grid=(B,),
        in_specs=[
            pl.BlockSpec((1, S, D), lambda i: (i, 0, 0)),
            pl.BlockSpec((1, S, D), lambda i: (i, 0, 0)),
            pl.BlockSpec((1, S, D), lambda i: (i, 0, 0)),
            pl.BlockSpec((1, S, D), lambda i: (i, 0, 0)),
            pl.BlockSpec((3, D), lambda i: (0, 0)),
            pl.BlockSpec((D, D // 8), lambda i: (0, 0)),
            pl.BlockSpec((1, D // 8), lambda i: (0, 0)),
            pl.BlockSpec((D // 8, D), lambda i: (0, 0)),
            pl.BlockSpec((1, D), lambda i: (0, 0)),
            pl.BlockSpec((2, D, D), lambda i: (0, 0, 0)),
            pl.BlockSpec((1, D), lambda i: (0, 0)),
            pl.BlockSpec((1, D), lambda i: (0, 0)),
            pl.BlockSpec((1, D), lambda i: (0, 0)),
        ],
        out_specs=pl.BlockSpec((1, S, D), lambda i: (i, 0, 0)),
        out_shape=jax.ShapeDtypeStruct((B, S, D), jnp.float32),
    )(x.reshape(B, S, D), h0.reshape(B, S, D), h1.reshape(B, S, D),
      h2.reshape(B, S, D), wrow, se_w1, se_b1.reshape(1, -1),
      se_w2, se_b2.reshape(1, -1), wf2, bf.reshape(1, -1),
      ln_g.reshape(1, -1), ln_b.reshape(1, -1))
    return out.reshape(N, D)


def kernel(x, edge_index, Wg, att_src, att_dst, bg, Wq, bq, Wk, bk, Wv, bv,
           Ws, bs, hop_w, se_w1, se_b1, se_w2, se_b2, Wf, bf, ln_g, ln_b):
    f32 = jnp.float32
    isq = 1.0 / jnp.sqrt(jnp.asarray(C, f32))
    src = edge_index[0]
    dst = edge_index[1]
    pad_e = jnp.full((EP - E,), N, jnp.int32)
    sidx = jnp.concatenate([src.astype(jnp.int32), pad_e])
    didx = jnp.concatenate([dst.astype(jnp.int32), pad_e])

    xp = jnp.zeros((NP, D), f32).at[:N].set(x)

    ar = jnp.arange(D)
    smask = (ar[:, None] // C == jnp.arange(16)[None, :]).astype(f32)
    emask = (jnp.arange(16)[:, None] == ar[None, :] // C).astype(f32)
    emask4 = emask[:H]
    zeros_v = jnp.zeros((RPT, D), f32)
    zeros_s = jnp.zeros((SROWS, D), f32)

    hops = []
    xi = xp
    for i in range(HOPS):
        wcat = jnp.concatenate(
            [Wg[i], Wq[i] * isq, Wk[i], Wv[i], Ws[i]], axis=1)
        bcat = jnp.concatenate(
            [jnp.zeros((D,), f32), bq[i] * isq, bk[i], bv[i], bs[i]]).reshape(1, -1)
        asrc_flat = att_src[i].reshape(1, D)
        adst_flat = att_dst[i].reshape(1, D)

        h_tab, ald_tab, ckv_tab, dq_tab, skip = _hop_pre(
            xi, wcat, bcat, asrc_flat, adst_flat, smask)
        als_tab = h_tab[:, 128:144]
        ald_tab16 = ald_tab[:, 0:16]

        als4 = als_tab[:, 0:4]
        ald4 = ald_tab[:, 0:4]
        zb = als4.max(axis=0) + ald4.max(axis=0)
        gb = jnp.where(zb > 0, zb, 0.2 * zb)
        qn = jnp.sqrt((dq_tab.reshape(NP, H, C) ** 2).sum(-1)).max(axis=0)
        kn = jnp.sqrt((ckv_tab[:, :D].reshape(NP, H, C) ** 2).sum(-1)).max(axis=0)
        tb = qn * kn
        gb_sc = jnp.zeros((8, 16), f32).at[0:4].set(
            jnp.broadcast_to(gb[:, None], (4, 16)))
        gb_sc = gb_sc.at[4, 0:4].set(gb)
        tb_sc = jnp.zeros((8, 16), f32).at[0:4].set(
            jnp.broadcast_to(tb[:, None], (4, 16)))
        gb_col = gb[ar // C].reshape(1, D)

        accg_v, pg = _gat_edge(sidx, didx, h_tab, ald_tab, gb_sc, zeros_v)
        acct_v, pt = _trans_edge(sidx, didx, ckv_tab, dq_tab, tb_sc, zeros_v)
        sg_out, st_out = _denom(didx, pg, pt, zeros_s)

        xi = _hop_post(accg_v, sg_out, acct_v, st_out, h_tab[:, 0:128],
                       als_tab, ald_tab16, skip, emask, emask4, gb_col,
                       bg[i].reshape(1, D))
        hops.append(xi)

    w = jax.nn.softmax(hop_w).astype(f32)
    wrow = jnp.broadcast_to(w[:, None], (3, D))
    return _epilogue(x, hops[0][:N], hops[1][:N], hops[2][:N], wrow,
                     se_w1, se_b1, se_w2, se_b2, Wf, bf, ln_g, ln_b)
